# Initial kernel scaffold; baseline (speedup 1.0000x reference)
import jax
import jax.numpy as jnp
from jax.experimental import pallas as pl


def kernel(atomic_numbers, emb):
    raise NotImplementedError("write your pallas kernel here")


# speedup vs baseline: 1.6063x; 1.6063x over previous
import jax
import jax.numpy as jnp
from jax import lax
from jax.experimental import pallas as pl
from jax.experimental.pallas import tpu as pltpu, tpu_sc as plsc

NUM_NODES = 100000
NUM_TYPES = 119
EMB_DIM = 128

NUM_CORES = 2
NUM_SUBCORES = 16
NUM_WORKERS = NUM_CORES * NUM_SUBCORES
CHUNK = 800
NCHUNKS = NUM_NODES // CHUNK
CHUNKS_PER_WORKER = -(-NCHUNKS // NUM_WORKERS)


def _emb_lookup_body(table_hbm, idx_hbm, out_hbm, idx_v, rows_v, sem):
    wid = lax.axis_index("s") * NUM_CORES + lax.axis_index("c")
    for j in range(CHUNKS_PER_WORKER):
        c = wid + NUM_WORKERS * j

        @pl.when(c < NCHUNKS)
        def _():
            base = c * CHUNK
            pltpu.sync_copy(idx_hbm.at[pl.ds(base, CHUNK)], idx_v)
            pltpu.async_copy(table_hbm.at[idx_v], rows_v, sem).wait()
            pltpu.sync_copy(rows_v, out_hbm.at[pl.ds(base, CHUNK)])


def kernel(atomic_numbers, emb):
    idx = atomic_numbers.astype(jnp.int32)
    mesh = plsc.VectorSubcoreMesh(
        core_axis_name="c", subcore_axis_name="s",
        num_cores=NUM_CORES, num_subcores=NUM_SUBCORES)
    out = pl.kernel(
        _emb_lookup_body,
        out_type=jax.ShapeDtypeStruct((NUM_NODES, EMB_DIM), jnp.float32),
        mesh=mesh,
        scratch_types=[
            pltpu.VMEM((CHUNK,), jnp.int32),
            pltpu.VMEM((CHUNK, EMB_DIM), jnp.float32),
            pltpu.SemaphoreType.DMA,
        ],
    )(emb, idx)
    return (out, out)
